# baseline (speedup 1.0000x reference)
import functools

import jax
import jax.numpy as jnp
from jax.experimental import pallas as pl

_N_NEWTON = 4
_N_SECANT = 3


def _simplex_block_kernel(x_ref, o_ref):
    x = x_ref[...]
    theta = jnp.max(x, axis=-1, keepdims=True) - 1.0
    prev_t = theta
    prev_f = jnp.zeros_like(theta)
    for _ in range(_N_NEWTON):
        mf = jnp.where(x > theta, 1.0, 0.0)
        s = jnp.sum(x * mf, axis=-1, keepdims=True)
        k = jnp.sum(mf, axis=-1, keepdims=True)
        prev_t = theta
        prev_f = s - k * theta - 1.0
        theta = (s - 1.0) / jnp.maximum(k, 1.0)
    for _ in range(_N_SECANT):
        f = jnp.sum(jnp.maximum(x - theta, 0.0), axis=-1, keepdims=True) - 1.0
        denom = prev_f - f
        step = jnp.where(
            denom > 0.0,
            f * (theta - prev_t) / jnp.where(denom == 0.0, 1.0, denom),
            0.0,
        )
        prev_t = theta
        prev_f = f
        theta = theta + jnp.maximum(step, 0.0)
    o_ref[...] = jnp.maximum(x - theta, 0.0)


@functools.partial(jax.jit, static_argnames=("block_rows", "interpret"))
def _project(x, block_rows=256, interpret=False):
    rows, n = x.shape
    grid = (rows // block_rows,)
    return pl.pallas_call(
        _simplex_block_kernel,
        grid=grid,
        in_specs=[pl.BlockSpec((block_rows, n), lambda i: (i, 0))],
        out_specs=pl.BlockSpec((block_rows, n), lambda i: (i, 0)),
        out_shape=jax.ShapeDtypeStruct((rows, n), x.dtype),
        interpret=interpret,
    )(x)


def kernel(x):
    return _project(x, block_rows=128)


# speedup vs baseline: 1.0968x; 1.0968x over previous
import functools

import jax
import jax.numpy as jnp
from jax.experimental import pallas as pl

_N_NEWTON = 4
_N_SECANT = 2


def _simplex_block_kernel(x_ref, o_ref):
    x = x_ref[...]
    theta = jnp.max(x, axis=-1, keepdims=True) - 1.0
    prev_t = theta
    prev_f = jnp.zeros_like(theta)
    for _ in range(_N_NEWTON):
        mf = jnp.where(x > theta, 1.0, 0.0)
        s = jnp.sum(x * mf, axis=-1, keepdims=True)
        k = jnp.sum(mf, axis=-1, keepdims=True)
        prev_t = theta
        prev_f = s - k * theta - 1.0
        theta = (s - 1.0) / jnp.maximum(k, 1.0)
    for _ in range(_N_SECANT):
        f = jnp.sum(jnp.maximum(x - theta, 0.0), axis=-1, keepdims=True) - 1.0
        denom = prev_f - f
        step = jnp.where(
            denom > 0.0,
            f * (theta - prev_t) / jnp.where(denom == 0.0, 1.0, denom),
            0.0,
        )
        prev_t = theta
        prev_f = f
        theta = theta + jnp.maximum(step, 0.0)
    o_ref[...] = jnp.maximum(x - theta, 0.0)


@functools.partial(jax.jit, static_argnames=("block_rows", "interpret"))
def _project(x, block_rows=256, interpret=False):
    rows, n = x.shape
    grid = (rows // block_rows,)
    return pl.pallas_call(
        _simplex_block_kernel,
        grid=grid,
        in_specs=[pl.BlockSpec((block_rows, n), lambda i: (i, 0))],
        out_specs=pl.BlockSpec((block_rows, n), lambda i: (i, 0)),
        out_shape=jax.ShapeDtypeStruct((rows, n), x.dtype),
        interpret=interpret,
    )(x)


def kernel(x):
    return _project(x, block_rows=256)
